# Initial kernel scaffold; baseline (speedup 1.0000x reference)
import jax
import jax.numpy as jnp
from jax.experimental import pallas as pl


def kernel(xt, G_c_t, G_s_t, t, kappa_t):
    raise NotImplementedError("write your pallas kernel here")


# speedup vs baseline: 1.9456x; 1.9456x over previous
import dataclasses
import functools
import math

import jax
import jax.numpy as jnp
from jax import lax
from jax.experimental import pallas as pl
from jax.experimental.pallas import tpu as pltpu
from jax.experimental.pallas import tpu_sc as plsc

_TWO_PI = 2.0 * math.pi
_HALF_PI = 0.5 * math.pi
_PI = math.pi

_SC_BLOCK = 2048
_TC_ROWS = 512


def _sc_gather(t, kappa_vec):
    n = t.shape[0]
    mesh = plsc.VectorSubcoreMesh(core_axis_name="c", subcore_axis_name="s")
    sc_params = pltpu.CompilerParams()
    if "needs_layout_passes" in pltpu.CompilerParams.__dataclass_fields__:
        sc_params = dataclasses.replace(sc_params, needs_layout_passes=False)

    @functools.partial(
        pl.kernel,
        mesh=mesh,
        compiler_params=sc_params,
        out_type=jax.ShapeDtypeStruct((1, n), jnp.float32),
        scratch_types=[pltpu.VMEM(kappa_vec.shape, jnp.float32)],
    )
    def gather_kernel(t_hbm, kappa_hbm, a_hbm, kappa_v):
        pltpu.sync_copy(kappa_hbm, kappa_v)

        def body(t_vmem, a_vmem):
            @pl.loop(0, _SC_BLOCK, step=16)
            def _(i):
                idx = t_vmem[0, pl.ds(i, 16)]
                a_vmem[0, pl.ds(i, 16)] = plsc.load_gather(kappa_v, [idx])

        pltpu.emit_pipeline(
            body,
            grid=(n // _SC_BLOCK,),
            in_specs=[pl.BlockSpec((1, _SC_BLOCK), lambda i: (0, i))],
            out_specs=[pl.BlockSpec((1, _SC_BLOCK), lambda i: (0, i))],
            core_axis_name=("c", "s"),
            dimension_semantics=(pltpu.PARALLEL,),
        )(t_hbm, a_hbm)

    return gather_kernel(t.reshape(1, n), kappa_vec)


_SIN_C = (1.0, -1.0 / 6, 1.0 / 120, -1.0 / 5040, 1.0 / 362880, -1.0 / 39916800.0,
          1.0 / 6227020800.0, -1.0 / 1307674368000.0, 1.0 / 355687428096000.0)
_COS_C = (1.0, -0.5, 1.0 / 24, -1.0 / 720, 1.0 / 40320, -1.0 / 3628800.0,
          1.0 / 479001600.0, -1.0 / 87178291200.0, 1.0 / 20922789888000.0)
_ATAN_C = (0.9998660, -0.3302995, 0.1801410, -0.0851330, 0.0208351)


def _poly(z, coeffs):
    p = jnp.float32(coeffs[-1])
    for c in coeffs[-2::-1]:
        p = p * z + jnp.float32(c)
    return p


def _tc_body(x_ref, gc_ref, gs_ref, a_ref, o_ref):
    a = a_ref[...]
    ki = lax.broadcasted_iota(jnp.int32, (128, 384), 0)
    ji = lax.broadcasted_iota(jnp.int32, (128, 384), 1)
    expand = jnp.where((ji >= 3 * ki) & (ji < 3 * ki + 3), 1.0, 0.0)
    arep = jnp.dot(a, expand, preferred_element_type=jnp.float32)

    w = jnp.float32(_TWO_PI) * x_ref[...] - jnp.float32(_PI)
    z = w * w
    sin_w = w * _poly(z, _SIN_C)
    cos_w = _poly(z, _COS_C)
    cos_gt = arep * cos_w + gc_ref[...]
    sin_gt = arep * sin_w + gs_ref[...]

    ax = jnp.abs(cos_gt)
    ay = jnp.abs(sin_gt)
    hi = jnp.maximum(ax, ay)
    lo = jnp.minimum(ax, ay)
    m = lo / jnp.maximum(hi, jnp.float32(1e-37))
    r = m * _poly(m * m, _ATAN_C)
    r = jnp.where(ay > ax, jnp.float32(_HALF_PI) - r, r)
    r = jnp.where(cos_gt < 0, jnp.float32(_PI) - r, r)
    theta = jnp.where(sin_gt < 0, -r, r)

    o_ref[0, :, :] = cos_gt
    o_ref[1, :, :] = sin_gt
    o_ref[2, :, :] = theta * jnp.float32(1.0 / _TWO_PI) + jnp.float32(0.5)


def kernel(xt, G_c_t, G_s_t, t, kappa_t):
    n = xt.shape[0]
    rows = (n * 3) // 384
    a = _sc_gather(t.astype(jnp.int32), kappa_t.reshape(-1))

    x2 = xt.reshape(rows, 384)
    gc2 = G_c_t.reshape(rows, 384)
    gs2 = G_s_t.reshape(rows, 384)
    a2 = a.reshape(rows, 128)

    grid = (rows // _TC_ROWS,)
    out = pl.pallas_call(
        _tc_body,
        grid=grid,
        in_specs=[
            pl.BlockSpec((_TC_ROWS, 384), lambda i: (i, 0)),
            pl.BlockSpec((_TC_ROWS, 384), lambda i: (i, 0)),
            pl.BlockSpec((_TC_ROWS, 384), lambda i: (i, 0)),
            pl.BlockSpec((_TC_ROWS, 128), lambda i: (i, 0)),
        ],
        out_specs=pl.BlockSpec((3, _TC_ROWS, 384), lambda i: (0, i, 0)),
        out_shape=jax.ShapeDtypeStruct((3, rows, 384), jnp.float32),
        compiler_params=pltpu.CompilerParams(
            dimension_semantics=("arbitrary",),
        ),
    )(x2, gc2, gs2, a2)
    return out.reshape(3, n, 3)
